# Initial kernel scaffold; baseline (speedup 1.0000x reference)
import jax
import jax.numpy as jnp
from jax.experimental import pallas as pl


def kernel(x, table):
    raise NotImplementedError("write your pallas kernel here")


# speedup vs baseline: 2.0530x; 2.0530x over previous
import functools

import jax
import jax.numpy as jnp
from jax import lax
from jax.experimental import pallas as pl
from jax.experimental.pallas import tpu as pltpu
from jax.experimental.pallas import tpu_sc as plsc


def _broadcast_body(num_cores, rows_per_w, batch, table_hbm, out_hbm, buf, sem):
    wid = lax.axis_index("s") * num_cores + lax.axis_index("c")
    base = wid * rows_per_w
    pltpu.sync_copy(table_hbm.at[pl.ds(base, rows_per_w)], buf)
    copies = [
        pltpu.async_copy(buf, out_hbm.at[b, pl.ds(base, rows_per_w)], sem)
        for b in range(batch)
    ]
    for cp in copies:
        cp.wait()


@functools.cache
def _make_broadcast(batch, num_rows, d_model, dtype):
    info = plsc.get_sparse_core_info()
    num_workers = info.num_cores * info.num_subcores
    assert num_rows % num_workers == 0
    rows_per_w = num_rows // num_workers
    mesh = plsc.VectorSubcoreMesh(core_axis_name="c", subcore_axis_name="s")
    return pl.kernel(
        functools.partial(_broadcast_body, info.num_cores, rows_per_w, batch),
        mesh=mesh,
        out_type=jax.ShapeDtypeStruct((batch, num_rows, d_model), dtype),
        scratch_types=[
            pltpu.VMEM((rows_per_w, d_model), dtype),
            pltpu.SemaphoreType.DMA,
        ],
    )


def kernel(x, table):
    batch, seq_len = x.shape
    num_rows, d_model = table.shape
    fn = _make_broadcast(batch, seq_len, d_model, table.dtype)
    return fn(table[:seq_len])


# speedup vs baseline: 2.0716x; 1.0091x over previous
import functools

import jax
import jax.numpy as jnp
from jax import lax
from jax.experimental import pallas as pl
from jax.experimental.pallas import tpu as pltpu
from jax.experimental.pallas import tpu_sc as plsc


_N_CHUNKS = 4


def _broadcast_body(num_cores, rows_per_w, batch, table_hbm, out_hbm, *rest):
    bufs, (rsem, wsem) = rest[:_N_CHUNKS], rest[_N_CHUNKS:]
    chunk = rows_per_w // _N_CHUNKS
    wid = lax.axis_index("s") * num_cores + lax.axis_index("c")
    base = wid * rows_per_w
    reads = [
        pltpu.async_copy(table_hbm.at[pl.ds(base + c * chunk, chunk)], bufs[c], rsem)
        for c in range(_N_CHUNKS)
    ]
    writes = []
    for c in range(_N_CHUNKS):
        reads[c].wait()
        writes += [
            pltpu.async_copy(
                bufs[c], out_hbm.at[b, pl.ds(base + c * chunk, chunk)], wsem
            )
            for b in range(batch)
        ]
    for cp in writes:
        cp.wait()


@functools.cache
def _make_broadcast(batch, num_rows, d_model, dtype):
    info = plsc.get_sparse_core_info()
    num_workers = info.num_cores * info.num_subcores
    assert num_rows % (num_workers * _N_CHUNKS) == 0
    rows_per_w = num_rows // num_workers
    mesh = plsc.VectorSubcoreMesh(core_axis_name="c", subcore_axis_name="s")
    return pl.kernel(
        functools.partial(_broadcast_body, info.num_cores, rows_per_w, batch),
        mesh=mesh,
        out_type=jax.ShapeDtypeStruct((batch, num_rows, d_model), dtype),
        scratch_types=[
            *[
                pltpu.VMEM((rows_per_w // _N_CHUNKS, d_model), dtype)
                for _ in range(_N_CHUNKS)
            ],
            pltpu.SemaphoreType.DMA,
            pltpu.SemaphoreType.DMA,
        ],
    )


def kernel(x, table):
    batch, seq_len = x.shape
    num_rows, d_model = table.shape
    fn = _make_broadcast(batch, seq_len, d_model, table.dtype)
    return fn(table[:seq_len])
